# Initial kernel scaffold; baseline (speedup 1.0000x reference)
import jax
import jax.numpy as jnp
from jax.experimental import pallas as pl


def kernel(ids, emb_table, W1, b1, W2, b2):
    raise NotImplementedError("write your pallas kernel here")


# speedup vs baseline: 2.8586x; 2.8586x over previous
import jax
import jax.numpy as jnp
from jax.experimental import pallas as pl
from jax.experimental.pallas import tpu as pltpu

_SEQ = 8192
_DIM = 128
_HID = 256
_VPAD = 128


def _tc_body(ids_ref, tab_ref, w1_ref, b1_ref, w2_ref, b2_ref, out_ref):
    ids = ids_ref[...]
    iota = jax.lax.broadcasted_iota(jnp.int32, (_SEQ, _VPAD), 1)
    onehot = (ids == iota).astype(jnp.float32)
    counts = jnp.sum(onehot, axis=0, keepdims=True)
    avg = jnp.dot(counts, tab_ref[...],
                  preferred_element_type=jnp.float32,
                  precision=jax.lax.Precision.HIGHEST) * (1.0 / _SEQ)
    h = jnp.maximum(
        jnp.dot(avg, w1_ref[...], preferred_element_type=jnp.float32,
                precision=jax.lax.Precision.HIGHEST) + b1_ref[...], 0.0)
    out_ref[...] = jnp.dot(
        h, w2_ref[...], preferred_element_type=jnp.float32,
        precision=jax.lax.Precision.HIGHEST) + b2_ref[...]


def kernel(ids, emb_table, W1, b1, W2, b2):
    ids2 = ids.reshape(_SEQ, 1)
    tab = jnp.zeros((_VPAD, _DIM), jnp.float32).at[: emb_table.shape[0]].set(emb_table)
    out = pl.pallas_call(
        _tc_body,
        out_shape=jax.ShapeDtypeStruct((1, _HID), jnp.float32),
    )(ids2, tab, W1, b1.reshape(1, _HID), W2, b2.reshape(1, _HID))
    return out.reshape(_HID)
